# baseline (speedup 1.0000x reference)
import jax
import jax.numpy as jnp
from jax import lax
from jax.experimental import pallas as pl
from jax.experimental.pallas import tpu as pltpu
from jax.experimental.pallas import tpu_sc as plsc

_S = 14
_C = 80
_NB = 3
_CELLS = _S * _S
_B = 256
_N = 32
_LC = 5.0
_LN = 0.5

_TPS = 6
_PPT = 40
_PLANE = 224
_HB = _B // 2
_NG = _HB // 16
_SYB = 2


def _encode_body(bx_hbm, lt_hbm, tgt_hbm, boxes_v, labels_v, slab_v):
    c = lax.axis_index("c")
    s = lax.axis_index("s")

    @pl.when(s < _TPS)
    def _():
        pltpu.sync_copy(bx_hbm.at[:, pl.ds(_HB * c, _HB)], boxes_v)
        pltpu.sync_copy(lt_hbm.at[:, pl.ds(_HB * c, _HB)], labels_v)

        zero16 = jnp.zeros((16,), jnp.float32)

        def _zero(p, carry):
            for j in range(_HB // 16):
                slab_v[4 * _PPT + p, pl.ds(16 * j, 16)] = zero16
            return carry
        lax.fori_loop(0, 36, _zero, 0)

        lid = lax.broadcasted_iota(jnp.int32, (16,), 0)
        ones = jnp.ones((16,), jnp.float32)
        posq = s * 32
        psize = jnp.where(s == _TPS - 1, 36, 32)

        def row(r):
            return jnp.full((16,), r, jnp.int32)

        def _group(g, carry):
            blane = 16 * g + lid
            for n in range(_N):
                x1 = boxes_v[4 * n + 0, pl.ds(16 * g, 16)]
                y1 = boxes_v[4 * n + 1, pl.ds(16 * g, 16)]
                x2 = boxes_v[4 * n + 2, pl.ds(16 * g, 16)]
                y2 = boxes_v[4 * n + 3, pl.ds(16 * g, 16)]
                lab = labels_v[n, pl.ds(16 * g, 16)]
                x = (x1 + x2) * 0.5
                y = (y1 + y2) * 0.5
                w = x2 - x1
                h = y2 - y1
                jj = jnp.minimum((x * float(_S)).astype(jnp.int32), _S - 1)
                ii = jnp.minimum((y * float(_S)).astype(jnp.int32), _S - 1)
                jj = jnp.maximum(jj, 0)
                ii = jnp.maximum(ii, 0)
                xc = x * float(_S) - jj.astype(jnp.float32)
                yc = y * float(_S) - ii.astype(jnp.float32)
                ploc = ii * _S + jj - posq
                inr = (ploc >= 0) & (ploc < psize)
                ploc = jnp.clip(ploc, 0, 35)
                conf = plsc.load_gather(slab_v, [row(4 * _PPT) + ploc, blane])
                won = inr & (conf == 0.0)
                plsc.store_scatter(slab_v, [row(0) + ploc, blane], xc,
                                   mask=won)
                plsc.store_scatter(slab_v, [row(_PPT) + ploc, blane], yc,
                                   mask=won)
                plsc.store_scatter(slab_v, [row(2 * _PPT) + ploc, blane], w,
                                   mask=won)
                plsc.store_scatter(slab_v, [row(3 * _PPT) + ploc, blane], h,
                                   mask=won)
                plsc.store_scatter(slab_v, [row(4 * _PPT) + ploc, blane],
                                   lab.astype(jnp.float32) + ones, mask=won)
            return carry
        lax.fori_loop(0, _NG, _group, 0)

        @pl.when(s < _TPS - 1)
        def _():
            for r in range(5):
                pltpu.sync_copy(
                    slab_v.at[pl.ds(r * _PPT, 32)],
                    tgt_hbm.at[pl.ds(r * _PLANE + posq, 32),
                               pl.ds(_HB * c, _HB)])

        @pl.when(s == _TPS - 1)
        def _():
            for r in range(5):
                pltpu.sync_copy(
                    slab_v.at[pl.ds(r * _PPT, 40)],
                    tgt_hbm.at[pl.ds(r * _PLANE + 160, 40),
                               pl.ds(_HB * c, _HB)])


_ENCODE_CACHE = []


def _encode(bx, lt):
    if not _ENCODE_CACHE:
        _ENCODE_CACHE.append(pl.kernel(
            _encode_body,
            mesh=plsc.VectorSubcoreMesh(core_axis_name="c",
                                        subcore_axis_name="s"),
            out_type=jax.ShapeDtypeStruct((5 * _PLANE, _B), jnp.float32),
            scratch_types=[
                pltpu.VMEM((4 * _N, _HB), jnp.float32),
                pltpu.VMEM((_N, _HB), jnp.int32),
                pltpu.VMEM((5 * _PPT, _HB), jnp.float32),
            ],
            compiler_params=pltpu.CompilerParams(needs_layout_passes=False),
        ))
    return _ENCODE_CACHE[0](bx, lt)


def _loss_body(x_ref, t_ref, o_ref, acc_ref):
    i = pl.program_id(0)

    @pl.when(i == 0)
    def _():
        acc_ref[...] = jnp.zeros_like(acc_ref)

    for q in range(_SYB):
        _loss_row(x_ref, t_ref, acc_ref, q)

    @pl.when(i == pl.num_programs(0) - 1)
    def _():
        s_xy = jnp.sum(acc_ref[0]) * (_LC / _B)
        s_wh = jnp.sum(acc_ref[1]) * (_LC / _B)
        s_co = jnp.sum(acc_ref[2]) * (1.0 / _B)
        s_no = jnp.sum(acc_ref[3]) * (_LN / _B)
        s_cl = jnp.sum(acc_ref[4]) * (1.0 / _B)
        tot = s_xy + s_wh + s_co + s_no + s_cl
        rows = lax.broadcasted_iota(jnp.int32, (8, 128), 0)
        o = jnp.where(rows == 0, s_xy,
            jnp.where(rows == 1, s_wh,
            jnp.where(rows == 2, s_co,
            jnp.where(rows == 3, s_no,
            jnp.where(rows == 4, s_cl, tot)))))
        o_ref[...] = o


def _loss_row(x_ref, t_ref, acc_ref, q):
    t = t_ref[:, q]
    objm = t[4] > 0.0
    obj = jnp.where(objm, 1.0, 0.0)
    tx = jnp.where(objm, t[0], 0.0)
    ty = jnp.where(objm, t[1], 0.0)
    tw = jnp.where(objm, t[2], 0.0)
    th = jnp.where(objm, t[3], 0.0)
    lab = t[4] - 1.0

    co = lax.broadcasted_iota(jnp.int32, (_S, _C, _B), 1).astype(jnp.float32)
    oh = (co == lab[:, None, :]).astype(jnp.float32)

    px, py, pw, ph, cf, clsl = [], [], [], [], [], []
    for k in range(_NB):
        px.append(x_ref[q, :, k, 0, :])
        py.append(x_ref[q, :, k, 1, :])
        pw.append(x_ref[q, :, k, 2, :])
        ph.append(x_ref[q, :, k, 3, :])
        cf.append(x_ref[q, :, k, 4, :])
        d = x_ref[q, :, k, 5:5 + _C, :] - oh
        clsl.append(jnp.sum(d * d, axis=1))

    bx1 = tx - tw * 0.5
    bx2 = tx + tw * 0.5
    by1 = ty - th * 0.5
    by2 = ty + th * 0.5
    area_b = jnp.maximum(bx2 - bx1, 0.0) * jnp.maximum(by2 - by1, 0.0)
    ious = []
    for k in range(_NB):
        ax1 = px[k] - pw[k] * 0.5
        ax2 = px[k] + pw[k] * 0.5
        ay1 = py[k] - ph[k] * 0.5
        ay2 = py[k] + ph[k] * 0.5
        iw = jnp.maximum(jnp.minimum(ax2, bx2) - jnp.maximum(ax1, bx1), 0.0)
        ih = jnp.maximum(jnp.minimum(ay2, by2) - jnp.maximum(ay1, by1), 0.0)
        inter = iw * ih
        area_a = jnp.maximum(ax2 - ax1, 0.0) * jnp.maximum(ay2 - ay1, 0.0)
        ious.append(inter / (area_a + area_b - inter + 1e-6))
    i0, i1, i2 = ious
    r0 = (i0 >= i1) & (i0 >= i2)
    r1 = jnp.logical_not(r0) & (i1 >= i2)

    def sel(v):
        return jnp.where(r0, v[0], jnp.where(r1, v[1], v[2]))

    xb, yb, wb, hb, cb = sel(px), sel(py), sel(pw), sel(ph), sel(cf)
    lcls = sel(clsl)
    confsq = cf[0] * cf[0] + cf[1] * cf[1] + cf[2] * cf[2]

    lxy = (xb - tx) ** 2 + (yb - ty) ** 2
    lwh = ((jnp.sqrt(jnp.maximum(wb, 1e-6)) -
            jnp.sqrt(jnp.maximum(tw, 1e-6))) ** 2 +
           (jnp.sqrt(jnp.maximum(hb, 1e-6)) -
            jnp.sqrt(jnp.maximum(th, 1e-6))) ** 2)
    lco = (cb - 1.0) ** 2

    acc_ref[0, 0:_S] += obj * lxy
    acc_ref[1, 0:_S] += obj * lwh
    acc_ref[2, 0:_S] += obj * lco
    acc_ref[3, 0:_S] += (1.0 - obj) * confsq
    acc_ref[4, 0:_S] += obj * lcls


def _loss_call(pt, t4):
    return pl.pallas_call(
        _loss_body,
        grid=(_S // _SYB,),
        in_specs=[
            pl.BlockSpec((_SYB, _S, _NB, 5 + _C, _B),
                         lambda i: (i, 0, 0, 0, 0)),
            pl.BlockSpec((5, _SYB, _S, _B), lambda i: (0, i, 0, 0)),
        ],
        out_specs=pl.BlockSpec((8, 128), lambda i: (0, 0)),
        out_shape=jax.ShapeDtypeStruct((8, 128), jnp.float32),
        scratch_shapes=[pltpu.VMEM((8, 16, _B), jnp.float32)],
        compiler_params=pltpu.CompilerParams(
            dimension_semantics=("arbitrary",)),
    )(pt, t4)


def kernel(predictions, targets_boxes, targets_labels):
    pt = jnp.transpose(predictions, (1, 2, 3, 4, 0))
    bx = jnp.transpose(targets_boxes, (1, 2, 0)).reshape(4 * _N, _B)
    lt = jnp.transpose(targets_labels, (1, 0))
    tgt = _encode(bx, lt)
    t4 = tgt.reshape(5, 16, _S, _B)
    out = _loss_call(pt, t4)
    return out[5, 0], out[0:5, 0]


# speedup vs baseline: 1.0143x; 1.0143x over previous
import jax
import jax.numpy as jnp
from jax import lax
from jax.experimental import pallas as pl
from jax.experimental.pallas import tpu as pltpu
from jax.experimental.pallas import tpu_sc as plsc

_S = 14
_C = 80
_NB = 3
_CELLS = _S * _S
_B = 256
_N = 32
_LC = 5.0
_LN = 0.5

_TPS = 6
_PPT = 40
_PLANE = 224
_HB = _B // 2
_NG = _HB // 16
_SYB = 2


def _encode_body(bx_hbm, lt_hbm, tgt_hbm, boxes_v, labels_v, slab_v,
                 sem1, sem2):
    c = lax.axis_index("c")
    s = lax.axis_index("s")

    @pl.when(s < _TPS)
    def _():
        cp1 = pltpu.async_copy(bx_hbm.at[:, pl.ds(_HB * c, _HB)], boxes_v,
                               sem1)
        cp2 = pltpu.async_copy(lt_hbm.at[:, pl.ds(_HB * c, _HB)], labels_v,
                               sem2)

        zero16 = jnp.zeros((16,), jnp.float32)

        def _zero(p, carry):
            for j in range(_HB // 16):
                slab_v[4 * _PPT + p, pl.ds(16 * j, 16)] = zero16
            return carry
        lax.fori_loop(0, 36, _zero, 0)
        cp1.wait()
        cp2.wait()

        lid = lax.broadcasted_iota(jnp.int32, (16,), 0)
        ones = jnp.ones((16,), jnp.float32)
        posq = s * 32
        psize = jnp.where(s == _TPS - 1, 36, 32)

        def row(r):
            return jnp.full((16,), r, jnp.int32)

        def _group(g, carry):
            blane = 16 * g + lid
            for n in range(_N):
                x1 = boxes_v[4 * n + 0, pl.ds(16 * g, 16)]
                y1 = boxes_v[4 * n + 1, pl.ds(16 * g, 16)]
                x2 = boxes_v[4 * n + 2, pl.ds(16 * g, 16)]
                y2 = boxes_v[4 * n + 3, pl.ds(16 * g, 16)]
                lab = labels_v[n, pl.ds(16 * g, 16)]
                x = (x1 + x2) * 0.5
                y = (y1 + y2) * 0.5
                w = x2 - x1
                h = y2 - y1
                jj = jnp.minimum((x * float(_S)).astype(jnp.int32), _S - 1)
                ii = jnp.minimum((y * float(_S)).astype(jnp.int32), _S - 1)
                jj = jnp.maximum(jj, 0)
                ii = jnp.maximum(ii, 0)
                xc = x * float(_S) - jj.astype(jnp.float32)
                yc = y * float(_S) - ii.astype(jnp.float32)
                ploc = ii * _S + jj - posq
                inr = (ploc >= 0) & (ploc < psize)
                ploc = jnp.clip(ploc, 0, 35)
                conf = plsc.load_gather(slab_v, [row(4 * _PPT) + ploc, blane])
                won = inr & (conf == 0.0)
                plsc.store_scatter(slab_v, [row(0) + ploc, blane], xc,
                                   mask=won)
                plsc.store_scatter(slab_v, [row(_PPT) + ploc, blane], yc,
                                   mask=won)
                plsc.store_scatter(slab_v, [row(2 * _PPT) + ploc, blane], w,
                                   mask=won)
                plsc.store_scatter(slab_v, [row(3 * _PPT) + ploc, blane], h,
                                   mask=won)
                plsc.store_scatter(slab_v, [row(4 * _PPT) + ploc, blane],
                                   lab.astype(jnp.float32) + ones, mask=won)
            return carry
        lax.fori_loop(0, _NG, _group, 0)

        @pl.when(s < _TPS - 1)
        def _():
            for r in range(5):
                pltpu.sync_copy(
                    slab_v.at[pl.ds(r * _PPT, 32)],
                    tgt_hbm.at[pl.ds(r * _PLANE + posq, 32),
                               pl.ds(_HB * c, _HB)])

        @pl.when(s == _TPS - 1)
        def _():
            for r in range(5):
                pltpu.sync_copy(
                    slab_v.at[pl.ds(r * _PPT, 40)],
                    tgt_hbm.at[pl.ds(r * _PLANE + 160, 40),
                               pl.ds(_HB * c, _HB)])


_ENCODE_CACHE = []


def _encode(bx, lt):
    if not _ENCODE_CACHE:
        _ENCODE_CACHE.append(pl.kernel(
            _encode_body,
            mesh=plsc.VectorSubcoreMesh(core_axis_name="c",
                                        subcore_axis_name="s"),
            out_type=jax.ShapeDtypeStruct((5 * _PLANE, _B), jnp.float32),
            scratch_types=[
                pltpu.VMEM((4 * _N, _HB), jnp.float32),
                pltpu.VMEM((_N, _HB), jnp.int32),
                pltpu.VMEM((5 * _PPT, _HB), jnp.float32),
                pltpu.SemaphoreType.DMA,
                pltpu.SemaphoreType.DMA,
            ],
            compiler_params=pltpu.CompilerParams(needs_layout_passes=False),
        ))
    return _ENCODE_CACHE[0](bx, lt)


def _loss_body(x_ref, t_ref, o_ref, acc_ref):
    i = pl.program_id(0)

    @pl.when(i == 0)
    def _():
        acc_ref[...] = jnp.zeros_like(acc_ref)

    for q in range(_SYB):
        _loss_row(x_ref, t_ref, acc_ref, q)

    @pl.when(i == pl.num_programs(0) - 1)
    def _():
        s_xy = jnp.sum(acc_ref[0]) * (_LC / _B)
        s_wh = jnp.sum(acc_ref[1]) * (_LC / _B)
        s_co = jnp.sum(acc_ref[2]) * (1.0 / _B)
        s_no = jnp.sum(acc_ref[3]) * (_LN / _B)
        s_cl = jnp.sum(acc_ref[4]) * (1.0 / _B)
        tot = s_xy + s_wh + s_co + s_no + s_cl
        rows = lax.broadcasted_iota(jnp.int32, (8, 128), 0)
        o = jnp.where(rows == 0, s_xy,
            jnp.where(rows == 1, s_wh,
            jnp.where(rows == 2, s_co,
            jnp.where(rows == 3, s_no,
            jnp.where(rows == 4, s_cl, tot)))))
        o_ref[...] = o


def _loss_row(x_ref, t_ref, acc_ref, q):
    t = t_ref[:, q]
    objm = t[4] > 0.0
    obj = jnp.where(objm, 1.0, 0.0)
    tx = jnp.where(objm, t[0], 0.0)
    ty = jnp.where(objm, t[1], 0.0)
    tw = jnp.where(objm, t[2], 0.0)
    th = jnp.where(objm, t[3], 0.0)
    lab = t[4] - 1.0

    co = lax.broadcasted_iota(jnp.int32, (_S, _C, _B), 1).astype(jnp.float32)
    oh = (co == lab[:, None, :]).astype(jnp.float32)

    px, py, pw, ph, cf, clsl = [], [], [], [], [], []
    for k in range(_NB):
        px.append(x_ref[q, :, k, 0, :])
        py.append(x_ref[q, :, k, 1, :])
        pw.append(x_ref[q, :, k, 2, :])
        ph.append(x_ref[q, :, k, 3, :])
        cf.append(x_ref[q, :, k, 4, :])
        d = x_ref[q, :, k, 5:5 + _C, :] - oh
        clsl.append(jnp.sum(d * d, axis=1))

    bx1 = tx - tw * 0.5
    bx2 = tx + tw * 0.5
    by1 = ty - th * 0.5
    by2 = ty + th * 0.5
    area_b = jnp.maximum(bx2 - bx1, 0.0) * jnp.maximum(by2 - by1, 0.0)
    ious = []
    for k in range(_NB):
        ax1 = px[k] - pw[k] * 0.5
        ax2 = px[k] + pw[k] * 0.5
        ay1 = py[k] - ph[k] * 0.5
        ay2 = py[k] + ph[k] * 0.5
        iw = jnp.maximum(jnp.minimum(ax2, bx2) - jnp.maximum(ax1, bx1), 0.0)
        ih = jnp.maximum(jnp.minimum(ay2, by2) - jnp.maximum(ay1, by1), 0.0)
        inter = iw * ih
        area_a = jnp.maximum(ax2 - ax1, 0.0) * jnp.maximum(ay2 - ay1, 0.0)
        ious.append(inter / (area_a + area_b - inter + 1e-6))
    i0, i1, i2 = ious
    r0 = (i0 >= i1) & (i0 >= i2)
    r1 = jnp.logical_not(r0) & (i1 >= i2)

    def sel(v):
        return jnp.where(r0, v[0], jnp.where(r1, v[1], v[2]))

    xb, yb, wb, hb, cb = sel(px), sel(py), sel(pw), sel(ph), sel(cf)
    lcls = sel(clsl)
    confsq = cf[0] * cf[0] + cf[1] * cf[1] + cf[2] * cf[2]

    lxy = (xb - tx) ** 2 + (yb - ty) ** 2
    lwh = ((jnp.sqrt(jnp.maximum(wb, 1e-6)) -
            jnp.sqrt(jnp.maximum(tw, 1e-6))) ** 2 +
           (jnp.sqrt(jnp.maximum(hb, 1e-6)) -
            jnp.sqrt(jnp.maximum(th, 1e-6))) ** 2)
    lco = (cb - 1.0) ** 2

    acc_ref[0, 0:_S] += obj * lxy
    acc_ref[1, 0:_S] += obj * lwh
    acc_ref[2, 0:_S] += obj * lco
    acc_ref[3, 0:_S] += (1.0 - obj) * confsq
    acc_ref[4, 0:_S] += obj * lcls


def _loss_call(pt, t4):
    return pl.pallas_call(
        _loss_body,
        grid=(_S // _SYB,),
        in_specs=[
            pl.BlockSpec((_SYB, _S, _NB, 5 + _C, _B),
                         lambda i: (i, 0, 0, 0, 0)),
            pl.BlockSpec((5, _SYB, _S, _B), lambda i: (0, i, 0, 0)),
        ],
        out_specs=pl.BlockSpec((8, 128), lambda i: (0, 0)),
        out_shape=jax.ShapeDtypeStruct((8, 128), jnp.float32),
        scratch_shapes=[pltpu.VMEM((8, 16, _B), jnp.float32)],
        compiler_params=pltpu.CompilerParams(
            dimension_semantics=("arbitrary",)),
    )(pt, t4)


def kernel(predictions, targets_boxes, targets_labels):
    pt = jnp.transpose(predictions, (1, 2, 3, 4, 0))
    bx = jnp.transpose(targets_boxes, (1, 2, 0)).reshape(4 * _N, _B)
    lt = jnp.transpose(targets_labels, (1, 0))
    tgt = _encode(bx, lt)
    t4 = tgt.reshape(5, 16, _S, _B)
    out = _loss_call(pt, t4)
    return out[5, 0], out[0:5, 0]
